# Initial kernel scaffold; baseline (speedup 1.0000x reference)
import jax
import jax.numpy as jnp
from jax.experimental import pallas as pl


def kernel(input_ids, segment_ids, W_word, W_pos, W_type, gamma, beta):
    raise NotImplementedError("write your pallas kernel here")


# speedup vs baseline: 1.8216x; 1.8216x over previous
import functools

import jax
import jax.numpy as jnp
from jax import lax
from jax.experimental import pallas as pl
from jax.experimental.pallas import tpu as pltpu
from jax.experimental.pallas import tpu_sc as plsc

NC = 2
NS = 16
NW = NC * NS


def _make_sc_gather(vocab, d, n, chunk):
    b_per_w = n // NW
    mesh = plsc.VectorSubcoreMesh(core_axis_name="c", subcore_axis_name="s")

    @functools.partial(
        pl.kernel,
        mesh=mesh,
        out_type=jax.ShapeDtypeStruct((n, d), jnp.float32),
        scratch_types=[
            pltpu.VMEM((chunk,), jnp.int32),
            pltpu.VMEM((chunk, d), jnp.float32),
            pltpu.SemaphoreType.DMA,
        ],
    )
    def sc_gather(table_hbm, idx_hbm, out_hbm, idx_v, rows_v, sem):
        wid = lax.axis_index("s") * NC + lax.axis_index("c")
        base = wid * b_per_w

        @pl.loop(0, b_per_w, step=chunk)
        def _(c):
            pltpu.sync_copy(idx_hbm.at[pl.ds(base + c, chunk)], idx_v)
            pltpu.async_copy(table_hbm.at[idx_v], rows_v, sem).wait()
            pltpu.sync_copy(rows_v, out_hbm.at[pl.ds(base + c, chunk)])

    return sc_gather


def _ln_body(g_ref, pos_ref, seg_ref, wt_ref, gamma_ref, beta_ref, o_ref):
    h = g_ref[...] + pos_ref[...]
    t0 = wt_ref[0:1, :]
    t1 = wt_ref[1:2, :]
    h = h + t0 + seg_ref[...] * (t1 - t0)
    mean = jnp.mean(h, axis=1, keepdims=True)
    c = h - mean
    var = jnp.mean(c * c, axis=1, keepdims=True)
    o_ref[...] = c * lax.rsqrt(var + 1e-12) * gamma_ref[...] + beta_ref[...]


def kernel(input_ids, segment_ids, W_word, W_pos, W_type, gamma, beta):
    b, s = input_ids.shape
    vocab, d = W_word.shape
    n = b * s

    ids_flat = input_ids.reshape(n).astype(jnp.int32)
    gathered = _make_sc_gather(vocab, d, n, chunk=64)(W_word, ids_flat)

    seg_f = segment_ids.reshape(n, 1).astype(jnp.float32)
    ts = 512
    nblk = s // ts
    out = pl.pallas_call(
        _ln_body,
        grid=(n // ts,),
        in_specs=[
            pl.BlockSpec((ts, d), lambda i: (i, 0)),
            pl.BlockSpec((ts, d), lambda i: (i % nblk, 0)),
            pl.BlockSpec((ts, 1), lambda i: (i, 0)),
            pl.BlockSpec((2, d), lambda i: (0, 0)),
            pl.BlockSpec((1, d), lambda i: (0, 0)),
            pl.BlockSpec((1, d), lambda i: (0, 0)),
        ],
        out_specs=pl.BlockSpec((ts, d), lambda i: (i, 0)),
        out_shape=jax.ShapeDtypeStruct((n, d), jnp.float32),
    )(gathered, W_pos, seg_f, W_type, gamma.reshape(1, d), beta.reshape(1, d))

    return out.reshape(b, s, d)


# speedup vs baseline: 1.9489x; 1.0699x over previous
import functools

import jax
import jax.numpy as jnp
from jax import lax
from jax.experimental import pallas as pl
from jax.experimental.pallas import tpu as pltpu
from jax.experimental.pallas import tpu_sc as plsc

NC = 2
NS = 16
NW = NC * NS


def _make_sc_gather(vocab, d, n, chunk):
    b_per_w = n // NW
    nch = b_per_w // chunk
    mesh = plsc.VectorSubcoreMesh(core_axis_name="c", subcore_axis_name="s")

    @functools.partial(
        pl.kernel,
        mesh=mesh,
        out_type=jax.ShapeDtypeStruct((n, d), jnp.float32),
        scratch_types=[
            pltpu.VMEM((b_per_w,), jnp.int32),
            pltpu.VMEM((chunk, d), jnp.float32),
            pltpu.VMEM((chunk, d), jnp.float32),
            pltpu.SemaphoreType.DMA,
            pltpu.SemaphoreType.DMA,
            pltpu.SemaphoreType.DMA,
            pltpu.SemaphoreType.DMA,
        ],
    )
    def sc_gather(table_hbm, idx_hbm, out_hbm, idx_v, rows0, rows1,
                  g0, g1, w0, w1):
        wid = lax.axis_index("s") * NC + lax.axis_index("c")
        base = wid * b_per_w
        pltpu.sync_copy(idx_hbm.at[pl.ds(base, b_per_w)], idx_v)
        rows = (rows0, rows1)
        gsem = (g0, g1)
        wsem = (w0, w1)
        gh = [None, None]
        wh = [None, None]
        for c in range(nch):
            bi = c & 1
            if wh[bi] is not None:
                wh[bi].wait()
            gh[bi] = pltpu.async_copy(
                table_hbm.at[idx_v.at[pl.ds(c * chunk, chunk)]],
                rows[bi], gsem[bi])
            if c >= 1:
                pj = 1 - bi
                gh[pj].wait()
                wh[pj] = pltpu.async_copy(
                    rows[pj],
                    out_hbm.at[pl.ds(base + (c - 1) * chunk, chunk)],
                    wsem[pj])
        last = (nch - 1) & 1
        gh[last].wait()
        wh[last] = pltpu.async_copy(
            rows[last],
            out_hbm.at[pl.ds(base + (nch - 1) * chunk, chunk)],
            wsem[last])
        wh[1 - last].wait()
        wh[last].wait()

    return sc_gather


def _ln_body(g_ref, pos_ref, seg_ref, wt_ref, gamma_ref, beta_ref, o_ref):
    h = g_ref[...] + pos_ref[...]
    t0 = wt_ref[0:1, :]
    t1 = wt_ref[1:2, :]
    h = h + t0 + seg_ref[...] * (t1 - t0)
    mean = jnp.mean(h, axis=1, keepdims=True)
    c = h - mean
    var = jnp.mean(c * c, axis=1, keepdims=True)
    o_ref[...] = c * lax.rsqrt(var + 1e-12) * gamma_ref[...] + beta_ref[...]


def kernel(input_ids, segment_ids, W_word, W_pos, W_type, gamma, beta):
    b, s = input_ids.shape
    vocab, d = W_word.shape
    n = b * s

    ids_flat = input_ids.reshape(n).astype(jnp.int32)
    gathered = _make_sc_gather(vocab, d, n, chunk=32)(W_word, ids_flat)

    seg_f = segment_ids.reshape(n, 1).astype(jnp.float32)
    ts = 512
    nblk = s // ts
    out = pl.pallas_call(
        _ln_body,
        grid=(nblk, b),
        in_specs=[
            pl.BlockSpec((ts, d), lambda j, i: (i * nblk + j, 0)),
            pl.BlockSpec((ts, d), lambda j, i: (j, 0)),
            pl.BlockSpec((ts, 1), lambda j, i: (i * nblk + j, 0)),
            pl.BlockSpec((2, d), lambda j, i: (0, 0)),
            pl.BlockSpec((1, d), lambda j, i: (0, 0)),
            pl.BlockSpec((1, d), lambda j, i: (0, 0)),
        ],
        out_specs=pl.BlockSpec((ts, d), lambda j, i: (i * nblk + j, 0)),
        out_shape=jax.ShapeDtypeStruct((n, d), jnp.float32),
    )(gathered, W_pos, seg_f, W_type, gamma.reshape(1, d), beta.reshape(1, d))

    return out.reshape(b, s, d)


# speedup vs baseline: 1.9984x; 1.0254x over previous
import functools

import jax
import jax.numpy as jnp
from jax import lax
from jax.experimental import pallas as pl
from jax.experimental.pallas import tpu as pltpu
from jax.experimental.pallas import tpu_sc as plsc

NC = 2
NS = 16
NW = NC * NS


def _make_sc_gather(d, n, chunk):
    b_per_w = n // NW
    nch = b_per_w // chunk
    mesh = plsc.VectorSubcoreMesh(core_axis_name="c", subcore_axis_name="s")

    @functools.partial(
        pl.kernel,
        mesh=mesh,
        out_type=jax.ShapeDtypeStruct((n, d), jnp.float32),
        scratch_types=[
            pltpu.VMEM((b_per_w,), jnp.int32),
            pltpu.VMEM((chunk, d), jnp.float32),
            pltpu.VMEM((chunk, d), jnp.float32),
            pltpu.SemaphoreType.DMA,
            pltpu.SemaphoreType.DMA,
            pltpu.SemaphoreType.DMA,
            pltpu.SemaphoreType.DMA,
        ],
    )
    def sc_gather(table_hbm, idx_hbm, out_hbm, idx_v, rows0, rows1,
                  g0, g1, w0, w1):
        wid = lax.axis_index("s") * NC + lax.axis_index("c")
        base = wid * b_per_w
        pltpu.sync_copy(idx_hbm.at[pl.ds(base, b_per_w)], idx_v)
        rows = (rows0, rows1)
        gsem = (g0, g1)
        wsem = (w0, w1)
        gh = [None, None]
        wh = [None, None]
        for c in range(nch):
            bi = c & 1
            if wh[bi] is not None:
                wh[bi].wait()
            gh[bi] = pltpu.async_copy(
                table_hbm.at[idx_v.at[pl.ds(c * chunk, chunk)]],
                rows[bi], gsem[bi])
            if c >= 1:
                pj = 1 - bi
                gh[pj].wait()
                wh[pj] = pltpu.async_copy(
                    rows[pj],
                    out_hbm.at[pl.ds(base + (c - 1) * chunk, chunk)],
                    wsem[pj])
        last = (nch - 1) & 1
        gh[last].wait()
        wh[last] = pltpu.async_copy(
            rows[last],
            out_hbm.at[pl.ds(base + (nch - 1) * chunk, chunk)],
            wsem[last])
        wh[1 - last].wait()
        wh[last].wait()

    return sc_gather


def _ln_math(g_ref, pos_ref, seg_ref, wt_ref, gamma_ref, beta_ref, o_ref):
    h = g_ref[...] + pos_ref[...]
    t0 = wt_ref[0:1, :]
    t1 = wt_ref[1:2, :]
    h = h + t0 + seg_ref[...] * (t1 - t0)
    mean = jnp.mean(h, axis=1, keepdims=True)
    c = h - mean
    var = jnp.mean(c * c, axis=1, keepdims=True)
    o_ref[...] = c * lax.rsqrt(var + 1e-12) * gamma_ref[...] + beta_ref[...]


def _ln_body_first(g_ref, pos_ref, seg_ref, wt_ref, gamma_ref, beta_ref,
                   o_ref):
    _ln_math(g_ref, pos_ref, seg_ref, wt_ref, gamma_ref, beta_ref, o_ref)


def _ln_body_alias(buf_ref, g_ref, pos_ref, seg_ref, wt_ref, gamma_ref,
                   beta_ref, o_ref):
    del buf_ref
    _ln_math(g_ref, pos_ref, seg_ref, wt_ref, gamma_ref, beta_ref, o_ref)


def kernel(input_ids, segment_ids, W_word, W_pos, W_type, gamma, beta):
    b, s = input_ids.shape
    vocab, d = W_word.shape
    n = b * s

    K = 4
    sk = s // K
    nk = b * sk
    ts = 512
    pb = sk // ts
    nblk = s // ts

    ids = input_ids.astype(jnp.int32)
    seg_f = segment_ids.astype(jnp.float32)
    gamma2 = gamma.reshape(1, d)
    beta2 = beta.reshape(1, d)

    sc_gather = _make_sc_gather(d, nk, chunk=32)

    staged = [sc_gather(W_word, ids[:, k * sk:(k + 1) * sk].reshape(nk))
              for k in range(K)]

    out_buf = None
    for k in range(K):
        seg_k = seg_f[:, k * sk:(k + 1) * sk].reshape(nk, 1)
        g_spec = pl.BlockSpec((ts, d), lambda j, i: (i * pb + j, 0))
        pos_spec = pl.BlockSpec((ts, d),
                                lambda j, i, k=k: (k * pb + j, 0))
        seg_spec = pl.BlockSpec((ts, 1), lambda j, i: (i * pb + j, 0))
        wt_spec = pl.BlockSpec((2, d), lambda j, i: (0, 0))
        vec_spec = pl.BlockSpec((1, d), lambda j, i: (0, 0))
        out_spec = pl.BlockSpec(
            (ts, d), lambda j, i, k=k: (i * nblk + k * pb + j, 0))
        if out_buf is None:
            out_buf = pl.pallas_call(
                _ln_body_first,
                grid=(pb, b),
                in_specs=[g_spec, pos_spec, seg_spec, wt_spec, vec_spec,
                          vec_spec],
                out_specs=out_spec,
                out_shape=jax.ShapeDtypeStruct((n, d), jnp.float32),
            )(staged[k], W_pos, seg_k, W_type, gamma2, beta2)
        else:
            out_buf = pl.pallas_call(
                _ln_body_alias,
                grid=(pb, b),
                in_specs=[pl.BlockSpec((8, 128), lambda j, i: (0, 0)),
                          g_spec, pos_spec, seg_spec, wt_spec, vec_spec,
                          vec_spec],
                out_specs=out_spec,
                out_shape=jax.ShapeDtypeStruct((n, d), jnp.float32),
                input_output_aliases={0: 0},
            )(out_buf, staged[k], W_pos, seg_k, W_type, gamma2, beta2)

    return out_buf.reshape(b, s, d)


# speedup vs baseline: 2.0648x; 1.0332x over previous
import functools

import jax
import jax.numpy as jnp
from jax import lax
from jax.experimental import pallas as pl
from jax.experimental.pallas import tpu as pltpu
from jax.experimental.pallas import tpu_sc as plsc

NC = 2
NS = 16
NW = NC * NS


def _make_sc_gather(d, b, s, k, sk, chunk):
    nk = b * sk
    b_per_w = nk // NW
    w_per_row = b * NW // (NW * b)
    wpb = NW // b
    nch = b_per_w // chunk
    mesh = plsc.VectorSubcoreMesh(core_axis_name="c", subcore_axis_name="s")

    @functools.partial(
        pl.kernel,
        mesh=mesh,
        out_type=jax.ShapeDtypeStruct((nk, d), jnp.float32),
        scratch_types=[
            pltpu.VMEM((b_per_w,), jnp.int32),
            pltpu.VMEM((chunk, d), jnp.float32),
            pltpu.VMEM((chunk, d), jnp.float32),
            pltpu.SemaphoreType.DMA,
            pltpu.SemaphoreType.DMA,
            pltpu.SemaphoreType.DMA,
            pltpu.SemaphoreType.DMA,
        ],
    )
    def sc_gather(table_hbm, idx_hbm, out_hbm, idx_v, rows0, rows1,
                  g0, g1, w0, w1):
        wid = lax.axis_index("s") * NC + lax.axis_index("c")
        b0 = wid // wpb
        p0 = k * sk + (wid % wpb) * b_per_w
        base = wid * b_per_w
        pltpu.sync_copy(idx_hbm.at[b0, pl.ds(p0, b_per_w)], idx_v)
        rows = (rows0, rows1)
        gsem = (g0, g1)
        wsem = (w0, w1)
        gh = [None, None]
        wh = [None, None]
        for c in range(nch):
            bi = c & 1
            if wh[bi] is not None:
                wh[bi].wait()
            gh[bi] = pltpu.async_copy(
                table_hbm.at[idx_v.at[pl.ds(c * chunk, chunk)]],
                rows[bi], gsem[bi])
            if c >= 1:
                pj = 1 - bi
                gh[pj].wait()
                wh[pj] = pltpu.async_copy(
                    rows[pj],
                    out_hbm.at[pl.ds(base + (c - 1) * chunk, chunk)],
                    wsem[pj])
        last = (nch - 1) & 1
        gh[last].wait()
        wh[last] = pltpu.async_copy(
            rows[last],
            out_hbm.at[pl.ds(base + (nch - 1) * chunk, chunk)],
            wsem[last])
        wh[1 - last].wait()
        wh[last].wait()

    return sc_gather


def _ln_math(g_ref, pos_ref, seg_ref, wt_ref, gamma_ref, beta_ref, o_ref):
    h = g_ref[...] + pos_ref[...]
    t0 = wt_ref[0:1, :]
    t1 = wt_ref[1:2, :]
    h = h + t0 + seg_ref[...] * (t1 - t0)
    mean = jnp.mean(h, axis=1, keepdims=True)
    c = h - mean
    var = jnp.mean(c * c, axis=1, keepdims=True)
    o_ref[...] = c * lax.rsqrt(var + 1e-12) * gamma_ref[...] + beta_ref[...]


def _ln_body_first(g_ref, pos_ref, seg_ref, wt_ref, gamma_ref, beta_ref,
                   o_ref):
    _ln_math(g_ref, pos_ref, seg_ref, wt_ref, gamma_ref, beta_ref, o_ref)


def _ln_body_alias(buf_ref, g_ref, pos_ref, seg_ref, wt_ref, gamma_ref,
                   beta_ref, o_ref):
    del buf_ref
    _ln_math(g_ref, pos_ref, seg_ref, wt_ref, gamma_ref, beta_ref, o_ref)


def kernel(input_ids, segment_ids, W_word, W_pos, W_type, gamma, beta):
    b, s = input_ids.shape
    vocab, d = W_word.shape
    n = b * s

    K = 4
    sk = s // K
    nk = b * sk
    ts = 1024
    pb = sk // ts
    nblk = s // ts

    ids = input_ids.astype(jnp.int32)
    seg_f = segment_ids.reshape(n, 1).astype(jnp.float32)
    gamma2 = gamma.reshape(1, d)
    beta2 = beta.reshape(1, d)

    staged = [_make_sc_gather(d, b, s, kk, sk, chunk=32)(W_word, ids)
              for kk in range(K)]

    out_buf = None
    for kk in range(K):
        g_spec = pl.BlockSpec((ts, d), lambda j, i: (i * pb + j, 0))
        pos_spec = pl.BlockSpec((ts, d),
                                lambda j, i, kk=kk: (kk * pb + j, 0))
        seg_spec = pl.BlockSpec(
            (ts, 1), lambda j, i, kk=kk: (i * nblk + kk * pb + j, 0))
        wt_spec = pl.BlockSpec((2, d), lambda j, i: (0, 0))
        vec_spec = pl.BlockSpec((1, d), lambda j, i: (0, 0))
        out_spec = pl.BlockSpec(
            (ts, d), lambda j, i, kk=kk: (i * nblk + kk * pb + j, 0))
        if out_buf is None:
            out_buf = pl.pallas_call(
                _ln_body_first,
                grid=(pb, b),
                in_specs=[g_spec, pos_spec, seg_spec, wt_spec, vec_spec,
                          vec_spec],
                out_specs=out_spec,
                out_shape=jax.ShapeDtypeStruct((n, d), jnp.float32),
            )(staged[kk], W_pos, seg_f, W_type, gamma2, beta2)
        else:
            out_buf = pl.pallas_call(
                _ln_body_alias,
                grid=(pb, b),
                in_specs=[pl.BlockSpec((8, 128), lambda j, i: (0, 0)),
                          g_spec, pos_spec, seg_spec, wt_spec, vec_spec,
                          vec_spec],
                out_specs=out_spec,
                out_shape=jax.ShapeDtypeStruct((n, d), jnp.float32),
                input_output_aliases={0: 0},
            )(out_buf, staged[kk], W_pos, seg_f, W_type, gamma2, beta2)

    return out_buf.reshape(b, s, d)


# speedup vs baseline: 2.3740x; 1.1498x over previous
import functools

import jax
import jax.numpy as jnp
from jax import lax
from jax.experimental import pallas as pl
from jax.experimental.pallas import tpu as pltpu
from jax.experimental.pallas import tpu_sc as plsc

NC = 2
NS = 16
NW = NC * NS


def _make_sc_gather(d, b, s, k, sk, chunk):
    nk = b * sk
    b_per_w = nk // NW
    w_per_row = b * NW // (NW * b)
    wpb = NW // b
    nch = b_per_w // chunk
    mesh = plsc.VectorSubcoreMesh(core_axis_name="c", subcore_axis_name="s")

    @functools.partial(
        pl.kernel,
        mesh=mesh,
        out_type=jax.ShapeDtypeStruct((nk, d), jnp.float32),
        scratch_types=[
            pltpu.VMEM((b_per_w,), jnp.int32),
            pltpu.VMEM((chunk, d), jnp.float32),
            pltpu.VMEM((chunk, d), jnp.float32),
            pltpu.SemaphoreType.DMA,
            pltpu.SemaphoreType.DMA,
            pltpu.SemaphoreType.DMA,
            pltpu.SemaphoreType.DMA,
        ],
    )
    def sc_gather(table_hbm, idx_hbm, out_hbm, idx_v, rows0, rows1,
                  g0, g1, w0, w1):
        wid = lax.axis_index("s") * NC + lax.axis_index("c")
        b0 = wid // wpb
        p0 = k * sk + (wid % wpb) * b_per_w
        base = wid * b_per_w
        pltpu.sync_copy(idx_hbm.at[b0, pl.ds(p0, b_per_w)], idx_v)
        rows = (rows0, rows1)
        gsem = (g0, g1)
        wsem = (w0, w1)
        gh = [None, None]
        wh = [None, None]
        for c in range(nch):
            bi = c & 1
            if wh[bi] is not None:
                wh[bi].wait()
            gh[bi] = pltpu.async_copy(
                table_hbm.at[idx_v.at[pl.ds(c * chunk, chunk)]],
                rows[bi], gsem[bi])
            if c >= 1:
                pj = 1 - bi
                gh[pj].wait()
                wh[pj] = pltpu.async_copy(
                    rows[pj],
                    out_hbm.at[pl.ds(base + (c - 1) * chunk, chunk)],
                    wsem[pj])
        last = (nch - 1) & 1
        gh[last].wait()
        wh[last] = pltpu.async_copy(
            rows[last],
            out_hbm.at[pl.ds(base + (nch - 1) * chunk, chunk)],
            wsem[last])
        wh[1 - last].wait()
        wh[last].wait()

    return sc_gather


def _ln_math(g_ref, pos_ref, seg_ref, wt_ref, gamma_ref, beta_ref, o_ref):
    h = g_ref[...] + pos_ref[...]
    t0 = wt_ref[0:1, :]
    t1 = wt_ref[1:2, :]
    h = h + t0 + seg_ref[...] * (t1 - t0)
    mean = jnp.mean(h, axis=1, keepdims=True)
    c = h - mean
    var = jnp.mean(c * c, axis=1, keepdims=True)
    o_ref[...] = c * lax.rsqrt(var + 1e-12) * gamma_ref[...] + beta_ref[...]


def _ln_body_first(g_ref, pos_ref, seg_ref, wt_ref, gamma_ref, beta_ref,
                   o_ref):
    _ln_math(g_ref, pos_ref, seg_ref, wt_ref, gamma_ref, beta_ref, o_ref)


def _ln_body_alias(buf_ref, g_ref, pos_ref, seg_ref, wt_ref, gamma_ref,
                   beta_ref, o_ref):
    del buf_ref
    _ln_math(g_ref, pos_ref, seg_ref, wt_ref, gamma_ref, beta_ref, o_ref)


def kernel(input_ids, segment_ids, W_word, W_pos, W_type, gamma, beta):
    b, s = input_ids.shape
    vocab, d = W_word.shape
    n = b * s

    K = 8
    sk = s // K
    nk = b * sk
    ts = 512
    pb = sk // ts
    nblk = s // ts

    ids = input_ids.astype(jnp.int32)
    seg_f = segment_ids.reshape(n, 1).astype(jnp.float32)
    gamma2 = gamma.reshape(1, d)
    beta2 = beta.reshape(1, d)

    staged = [_make_sc_gather(d, b, s, kk, sk, chunk=32)(W_word, ids)
              for kk in range(K)]

    out_buf = None
    for kk in range(K):
        g_spec = pl.BlockSpec((ts, d), lambda j, i: (i * pb + j, 0))
        pos_spec = pl.BlockSpec((ts, d),
                                lambda j, i, kk=kk: (kk * pb + j, 0))
        seg_spec = pl.BlockSpec(
            (ts, 1), lambda j, i, kk=kk: (i * nblk + kk * pb + j, 0))
        wt_spec = pl.BlockSpec((2, d), lambda j, i: (0, 0))
        vec_spec = pl.BlockSpec((1, d), lambda j, i: (0, 0))
        out_spec = pl.BlockSpec(
            (ts, d), lambda j, i, kk=kk: (i * nblk + kk * pb + j, 0))
        if out_buf is None:
            out_buf = pl.pallas_call(
                _ln_body_first,
                grid=(pb, b),
                in_specs=[g_spec, pos_spec, seg_spec, wt_spec, vec_spec,
                          vec_spec],
                out_specs=out_spec,
                out_shape=jax.ShapeDtypeStruct((n, d), jnp.float32),
            )(staged[kk], W_pos, seg_f, W_type, gamma2, beta2)
        else:
            out_buf = pl.pallas_call(
                _ln_body_alias,
                grid=(pb, b),
                in_specs=[pl.BlockSpec((8, 128), lambda j, i: (0, 0)),
                          g_spec, pos_spec, seg_spec, wt_spec, vec_spec,
                          vec_spec],
                out_specs=out_spec,
                out_shape=jax.ShapeDtypeStruct((n, d), jnp.float32),
                input_output_aliases={0: 0},
            )(out_buf, staged[kk], W_pos, seg_f, W_type, gamma2, beta2)

    return out_buf.reshape(b, s, d)
